# baseline (speedup 1.0000x reference)
import functools

import jax
import jax.numpy as jnp
from jax.experimental import pallas as pl
from jax.experimental.pallas import tpu as pltpu

S = 2048
D = 768
H = 2048
E = 8
K = 2
SB = 256
HB = 512


def _router_body(x_ref, gw_ref, gb_ref, wgt_ref, loss_ref):
    x = x_ref[...]
    gw = gw_ref[...]
    logits = jax.lax.dot_general(
        x, gw, (((1,), (1,)), ((), ())),
        preferred_element_type=jnp.float32) + gb_ref[...]

    r = jax.lax.broadcasted_iota(jnp.int32, (E, E), 0)
    c = jax.lax.broadcasted_iota(jnp.int32, (E, E), 1)
    lt = (r < c).astype(jnp.float32)

    def first_max_onehot(lg):
        m = jnp.max(lg, axis=1, keepdims=True)
        t = (lg == m).astype(jnp.float32)
        prefix = jax.lax.dot_general(t, lt, (((1,), (0,)), ((), ())),
                                     preferred_element_type=jnp.float32)
        return jnp.where((t > 0.0) & (prefix == 0.0), 1.0, 0.0), m

    oh0, m0 = first_max_onehot(logits)
    masked = jnp.where(oh0 > 0.0, -jnp.inf, logits)
    oh1, m1 = first_max_onehot(masked)

    p0 = 1.0 / (1.0 + jnp.exp(m1 - m0))
    p1 = 1.0 - p0
    wgt_ref[...] = oh0 * p0 + oh1 * p1

    z = jnp.exp(logits - m0)
    sm = z / jnp.sum(z, axis=1, keepdims=True)
    importance = jnp.sum(sm, axis=0, keepdims=True) / float(S)
    load = jnp.sum(oh0 + oh1, axis=0, keepdims=True) / float(S * K)
    loss_ref[...] = jnp.sum(importance * load, axis=1, keepdims=True) * float(E)


def _ffn_body(x_ref, w1_ref, b1_ref, w2_ref, b2_ref, w3_ref, b3_ref,
              wgt_ref, out_ref):
    e = pl.program_id(0)
    h = pl.program_id(1)
    s = pl.program_id(2)
    rows = pl.ds(s * SB, SB)
    xb = x_ref[rows, :]
    lane = jax.lax.broadcasted_iota(jnp.int32, (SB, E), 1)
    wcol = jnp.sum(wgt_ref[rows, :] * (lane == e).astype(jnp.float32),
                   axis=1, keepdims=True)

    a = jax.lax.dot_general(xb, w1_ref[0], (((1,), (1,)), ((), ())),
                            preferred_element_type=jnp.float32) + b1_ref[0]
    b = jax.lax.dot_general(xb, w2_ref[0], (((1,), (1,)), ((), ())),
                            preferred_element_type=jnp.float32) + b2_ref[0]
    hp = (a * jax.lax.logistic(a)) * b
    yp = jax.lax.dot_general(hp, w3_ref[0], (((1,), (1,)), ((), ())),
                             preferred_element_type=jnp.float32)

    @pl.when((e == 0) & (h == 0))
    def _init():
        out_ref[rows, :] = wcol * (yp + b3_ref[0])

    @pl.when((e > 0) & (h == 0))
    def _first_h():
        out_ref[rows, :] += wcol * (yp + b3_ref[0])

    @pl.when(h > 0)
    def _acc():
        out_ref[rows, :] += wcol * yp


@jax.jit
def _moe(x2d, gate_W, gb2d, W1, b1r, W2, b2r, W3, b3r):
    wgt, loss = pl.pallas_call(
        _router_body,
        out_shape=(jax.ShapeDtypeStruct((S, E), jnp.float32),
                   jax.ShapeDtypeStruct((1, 1), jnp.float32)),
    )(x2d, gate_W, gb2d)

    grid = (E, H // HB, S // SB)
    out = pl.pallas_call(
        _ffn_body,
        grid=grid,
        in_specs=[
            pl.BlockSpec((S, D), lambda e, h, s: (0, 0)),
            pl.BlockSpec((1, HB, D), lambda e, h, s: (e, h, 0)),
            pl.BlockSpec((1, 1, HB), lambda e, h, s: (e, 0, h)),
            pl.BlockSpec((1, HB, D), lambda e, h, s: (e, h, 0)),
            pl.BlockSpec((1, 1, HB), lambda e, h, s: (e, 0, h)),
            pl.BlockSpec((1, D, HB), lambda e, h, s: (e, 0, h)),
            pl.BlockSpec((1, 1, D), lambda e, h, s: (e, 0, 0)),
            pl.BlockSpec((S, E), lambda e, h, s: (0, 0)),
        ],
        out_specs=pl.BlockSpec((S, D), lambda e, h, s: (0, 0)),
        out_shape=jax.ShapeDtypeStruct((S, D), jnp.float32),
        compiler_params=pltpu.CompilerParams(
            dimension_semantics=("arbitrary", "arbitrary", "arbitrary")),
    )(x2d, W1, b1r, W2, b2r, W3, b3r, wgt)
    return out, loss


def kernel(x, gate_W, gate_b, W1, b1, W2, b2, W3, b3):
    x2d = x.reshape(S, D)
    gb2d = gate_b.reshape(1, E)
    b1r = b1.reshape(E, 1, H)
    b2r = b2.reshape(E, 1, H)
    b3r = b3.reshape(E, 1, D)
    out, loss = _moe(x2d, gate_W, gb2d, W1, b1r, W2, b2r, W3, b3r)
    return out.reshape(1, S, D), loss.reshape(())
